# baseline (speedup 1.0000x reference)
import functools
import math

import jax
import jax.numpy as jnp
from jax import lax
from jax.experimental import pallas as pl
from jax.experimental.pallas import tpu as pltpu
from jax.experimental.pallas import tpu_sc as plsc

_BN = 1000
_C = 48


def _proj_body(nt_ref, h_ref, kw_ref, qw_ref, vw_ref, ratt_ref, rmsg_ref,
               pri_ref, k_out, qwt_out, vwt_out):
    nt = nt_ref[...]
    h = h_ref[...]
    n_t = kw_ref.shape[0]
    n_r = ratt_ref.shape[0]
    k = jnp.zeros((h.shape[0], kw_ref.shape[2]), jnp.float32)
    q = jnp.zeros_like(k)
    v = jnp.zeros_like(k)
    for t in range(n_t):
        m = nt == t
        k = jnp.where(m, jnp.dot(h, kw_ref[t], preferred_element_type=jnp.float32), k)
        q = jnp.where(m, jnp.dot(h, qw_ref[t], preferred_element_type=jnp.float32), q)
        v = jnp.where(m, jnp.dot(h, vw_ref[t], preferred_element_type=jnp.float32), v)
    k_out[...] = k
    inv_sqrt_dk = 1.0 / math.sqrt(float(kw_ref.shape[2]))
    for r in range(n_r):
        pri = pri_ref[:, r:r + 1] * inv_sqrt_dk
        qwt_out[r] = jnp.dot(q, ratt_ref[r], preferred_element_type=jnp.float32) * pri
        vwt_out[r] = jnp.dot(v, rmsg_ref[r], preferred_element_type=jnp.float32)


def _proj(nt2, h, kw, qw, vw, ratt, rmsg, pri):
    n, in_dim = h.shape
    out_dim = kw.shape[2]
    n_t, n_r = kw.shape[0], ratt.shape[0]
    grid = (n // _BN,)
    return pl.pallas_call(
        _proj_body,
        grid=grid,
        in_specs=[
            pl.BlockSpec((_BN, 1), lambda i: (i, 0)),
            pl.BlockSpec((_BN, in_dim), lambda i: (i, 0)),
            pl.BlockSpec((n_t, in_dim, out_dim), lambda i: (0, 0, 0)),
            pl.BlockSpec((n_t, in_dim, out_dim), lambda i: (0, 0, 0)),
            pl.BlockSpec((n_t, in_dim, out_dim), lambda i: (0, 0, 0)),
            pl.BlockSpec((n_r, out_dim, out_dim), lambda i: (0, 0, 0)),
            pl.BlockSpec((n_r, out_dim, out_dim), lambda i: (0, 0, 0)),
            pl.BlockSpec((1, n_r), lambda i: (0, 0)),
        ],
        out_specs=[
            pl.BlockSpec((_BN, out_dim), lambda i: (i, 0)),
            pl.BlockSpec((n_r, _BN, out_dim), lambda i: (0, i, 0)),
            pl.BlockSpec((n_r, _BN, out_dim), lambda i: (0, i, 0)),
        ],
        out_shape=[
            jax.ShapeDtypeStruct((n, out_dim), jnp.float32),
            jax.ShapeDtypeStruct((n_r, n, out_dim), jnp.float32),
            jax.ShapeDtypeStruct((n_r, n, out_dim), jnp.float32),
        ],
    )(nt2, h, kw, qw, vw, ratt, rmsg, pri)


def _edgeidx_body(n_nodes, src_ref, dst_ref, et_ref, qwi_ref, ki_ref,
                  vwi_ref, di_ref):
    src = src_ref[...]
    dst = dst_ref[...]
    et = et_ref[...]
    qwi_ref[...] = et * n_nodes + dst
    ki_ref[...] = src
    vwi_ref[...] = et * n_nodes + src
    di_ref[...] = dst


def _edgeidx(src2, dst2, et2, n_nodes):
    rows, cols = src2.shape
    shp = jax.ShapeDtypeStruct((rows, cols), jnp.int32)
    return pl.pallas_call(
        functools.partial(_edgeidx_body, n_nodes),
        out_shape=[shp, shp, shp, shp],
    )(src2, dst2, et2)


def _edge_sc(qw_tab, k_tab, vw_tab, eidx, n_nodes):
    out_dim = k_tab.shape[1]
    info = plsc.get_sparse_core_info()
    nc, ns = info.num_cores, info.num_subcores
    nw = nc * ns
    nch = eidx.shape[0] // (nw * 4 * _C)
    nrd = 10
    rs = n_nodes // nrd
    zr = 10
    nzero = rs // zr
    ndr = (-(-n_nodes // out_dim) + 15) // 16 * 16
    mesh = plsc.VectorSubcoreMesh(core_axis_name="c", subcore_axis_name="s")

    @functools.partial(
        pl.kernel,
        out_type=[
            jax.ShapeDtypeStruct((nc, n_nodes, out_dim), jnp.float32),
            jax.ShapeDtypeStruct((nc, ndr, out_dim), jnp.float32),
        ],
        mesh=mesh,
        compiler_params=pltpu.CompilerParams(needs_layout_passes=False),
        scratch_types=[
            pltpu.VMEM((4, 2 * _C), jnp.int32),
            pltpu.VMEM((_C,), jnp.int32),
            pltpu.VMEM((_C,), jnp.int32),
            pltpu.VMEM((ndr,), jnp.int32),
            pltpu.VMEM((2 * _C, out_dim), jnp.float32),
            pltpu.VMEM((2 * _C, out_dim), jnp.float32),
            pltpu.VMEM((2 * _C, out_dim), jnp.float32),
            pltpu.VMEM((ndr * out_dim,), jnp.float32),
            pltpu.VMEM((_C,), jnp.float32),
            pltpu.VMEM((zr, out_dim), jnp.float32),
            pltpu.VMEM_SHARED((n_nodes + 8, out_dim), jnp.float32),
            pltpu.VMEM_SHARED((ndr, out_dim), jnp.float32),
            pltpu.SemaphoreType.DMA,
            pltpu.SemaphoreType.DMA,
            pltpu.SemaphoreType.DMA,
            pltpu.SemaphoreType.DMA,
            pltpu.SemaphoreType.DMA,
            pltpu.SemaphoreType.DMA,
        ],
    )
    def _k(qw_hbm, k_hbm, vw_hbm, eidx_hbm,
           agg_out, den_out,
           ibuf, sbuf0, sbuf1, idb, qwr, kr, vwr, denp, exv, zbuf, agg_sh,
           den_sh,
           semi0, semi1, semg0, semg1, sems0, sems1):
        cid = lax.axis_index("c")
        sid = lax.axis_index("s")
        wid = sid * nc + cid
        semi = (semi0, semi1)
        semg = (semg0, semg1)
        sems = (sems0, sems1)
        sbuf = (sbuf0, sbuf1)
        zf16 = jnp.zeros((16,), jnp.float32)
        iota16 = lax.iota(jnp.int32, 16)
        bfly = [(iota16 ^ sh)[:, None] for sh in (8, 4, 2, 1)]
        dn = lax.GatherDimensionNumbers(
            offset_dims=(), collapsed_slice_dims=(0,), start_index_map=(0,))

        def _hsum(x):
            for idx in bfly:
                x = x + lax.gather(x, idx, dn, slice_sizes=(1,),
                                   mode=lax.GatherScatterMode.PROMISE_IN_BOUNDS)
            return x

        def _zrow(i, _):
            r = i // (out_dim // 16)
            c = (i % (out_dim // 16)) * 16
            zbuf[r, pl.ds(c, 16)] = zf16
            return _
        lax.fori_loop(0, zr * (out_dim // 16), _zrow, None)

        def _zden(i, _):
            denp[pl.ds(i * 16, 16)] = zf16
            return _
        lax.fori_loop(0, ndr * out_dim // 16, _zden, None)

        for g in range(ndr // 16):
            idb[pl.ds(g * 16, 16)] = iota16 + g * 16

        @pl.when(sid < nrd)
        def _zero_shared():
            for g in range(nzero):
                pltpu.sync_copy(zbuf, agg_sh.at[pl.ds(sid * rs + g * zr, zr)])

        @pl.when(sid == nrd)
        def _zero_den():
            for g in range(ndr // zr):
                pltpu.sync_copy(zbuf, den_sh.at[pl.ds(g * zr, zr)])
        plsc.subcore_barrier()

        cbase = wid * nch

        def _idx_cps(g, b):
            base = (cbase + g) * (4 * _C)
            return [
                (eidx_hbm.at[pl.ds(base, 2 * _C)], ibuf.at[2 * b]),
                (eidx_hbm.at[pl.ds(base + 2 * _C, 2 * _C)], ibuf.at[2 * b + 1]),
            ]

        def _gath(b):
            base = b * _C
            return [
                (qw_hbm.at[ibuf.at[2 * b, pl.ds(0, _C)]],
                 qwr.at[pl.ds(base, _C)]),
                (k_hbm.at[ibuf.at[2 * b, pl.ds(_C, _C)]],
                 kr.at[pl.ds(base, _C)]),
                (vw_hbm.at[ibuf.at[2 * b + 1, pl.ds(0, _C)]],
                 vwr.at[pl.ds(base, _C)]),
            ]

        def _scat(b):
            return (vwr.at[pl.ds(b * _C, _C)], agg_sh.at[sbuf[b]])

        def _phase(g, b):
            nb = 1 - b

            @pl.when(g + 1 < nch)
            def _prefetch():
                for s, d in _idx_cps(g + 1, nb):
                    pltpu.make_async_copy(s, d, semi[nb]).wait()

                @pl.when(g >= 1)
                def _wait_scat():
                    s, d = _scat(nb)
                    pltpu.make_async_copy(s, d, sems[nb]).wait()
                for s, d in _gath(nb):
                    pltpu.async_copy(s, d, semg[nb])

            for s, d in _gath(b):
                pltpu.make_async_copy(s, d, semg[b]).wait()

            for grp in range(_C // 16):
                sbuf[b][pl.ds(grp * 16, 16)] = ibuf[2 * b + 1, pl.ds(_C + grp * 16, 16)]

            @pl.when(g + 2 < nch)
            def _prefetch_idx():
                for s, d in _idx_cps(g + 2, b):
                    pltpu.async_copy(s, d, semi[b])

            base = b * _C

            @plsc.parallel_loop(0, _C, 1, unroll=8)
            def _edge(i):
                e = base + i
                acc = qwr[e, pl.ds(0, 16)] * kr[e, pl.ds(0, 16)]
                for c in range(1, out_dim // 16):
                    acc = acc + qwr[e, pl.ds(c * 16, 16)] * kr[e, pl.ds(c * 16, 16)]
                exb = jnp.exp(_hsum(acc))
                plsc.store_scatter(exv, [jnp.full((16,), i, jnp.int32)], exb,
                                   mask=iota16 == 0)
                for c in range(out_dim // 16):
                    vwr[e, pl.ds(c * 16, 16)] = vwr[e, pl.ds(c * 16, 16)] * exb

            def _grp(grp, _):
                dstv = sbuf[b][pl.ds(grp * 16, 16)]
                exg = exv[pl.ds(grp * 16, 16)]

                def _den(j, _):
                    plsc.addupdate_scatter(denp, [dstv], exg, mask=iota16 == j)
                    return _
                lax.fori_loop(0, 16, _den, None)
                return _
            lax.fori_loop(0, _C // 16, _grp, None)
            s, d = _scat(b)
            pltpu.async_copy(s, d, sems[b], add=True)

        for s, d in _idx_cps(0, 0):
            pltpu.async_copy(s, d, semi0)
        for s, d in _idx_cps(0, 0):
            pltpu.make_async_copy(s, d, semi0).wait()
        for s, d in _gath(0):
            pltpu.async_copy(s, d, semg0)
        for s, d in _idx_cps(1, 1):
            pltpu.async_copy(s, d, semi1)

        def _outer(o, _):
            _phase(o * 2, 0)
            _phase(o * 2 + 1, 1)
            return _
        lax.fori_loop(0, nch // 2, _outer, None)
        _phase(nch - 1, 0)
        for b in (0, 1):
            s, d = _scat(b)
            pltpu.make_async_copy(s, d, sems[b]).wait()

        def _pack(i, _):
            r = i // (out_dim // 16)
            c = (i % (out_dim // 16)) * 16
            qwr[r, pl.ds(c, 16)] = denp[pl.ds(r * out_dim + c, 16)]
            return _
        lax.fori_loop(0, ndr * out_dim // 16, _pack, None)
        pltpu.sync_copy(qwr.at[pl.ds(0, ndr)], den_sh.at[idb], add=True)
        plsc.subcore_barrier()

        @pl.when(sid < nrd)
        def _readout():
            pltpu.sync_copy(agg_sh.at[pl.ds(sid * rs, rs)],
                            agg_out.at[cid, pl.ds(sid * rs, rs)])

        @pl.when(sid == nrd)
        def _readout_den():
            pltpu.sync_copy(den_sh, den_out.at[cid])

    return _k(qw_tab, k_tab, vw_tab, eidx)


def _final_body(nt_ref, agg_ref, den_ref, aw_ref, skip_ref, out_ref):
    nt = nt_ref[...]
    agg = agg_ref[0] + agg_ref[1]
    den = den_ref[0] + den_ref[1]
    rows = agg / (den + 1e-16)
    n_t = aw_ref.shape[0]
    out = jnp.zeros_like(rows)
    for t in range(n_t):
        sig = 1.0 / (1.0 + jnp.exp(-skip_ref[:, t:t + 1]))
        y = jnp.dot(rows, aw_ref[t], preferred_element_type=jnp.float32) * sig
        out = jnp.where(nt == t, y, out)
    out_ref[...] = out


def _final(nt2, agg2, den2, aw, skip_row):
    n = nt2.shape[0]
    out_dim = aw.shape[2]
    n_t = aw.shape[0]
    grid = (n // _BN,)
    return pl.pallas_call(
        _final_body,
        grid=grid,
        in_specs=[
            pl.BlockSpec((_BN, 1), lambda i: (i, 0)),
            pl.BlockSpec((2, _BN, out_dim), lambda i: (0, i, 0)),
            pl.BlockSpec((2, _BN, 1), lambda i: (0, i, 0)),
            pl.BlockSpec((n_t, out_dim, out_dim), lambda i: (0, 0, 0)),
            pl.BlockSpec((1, n_t), lambda i: (0, 0)),
        ],
        out_specs=pl.BlockSpec((_BN, out_dim), lambda i: (i, 0)),
        out_shape=jax.ShapeDtypeStruct((n, out_dim), jnp.float32),
    )(nt2, agg2, den2, aw, skip_row)


def kernel(h, edge_index, edge_type, node_type, k_linears, q_linears,
           v_linears, a_linears, relation_att, relation_msg, relation_pri,
           skip):
    n, in_dim = h.shape
    e = edge_index.shape[1]
    n_t = k_linears.shape[0]
    n_r = relation_att.shape[0]
    out_dim = k_linears.shape[3]

    kw = k_linears.reshape(n_t, in_dim, out_dim)
    qw = q_linears.reshape(n_t, in_dim, out_dim)
    vw = v_linears.reshape(n_t, in_dim, out_dim)
    ratt = relation_att.reshape(n_r, out_dim, out_dim)
    rmsg = relation_msg.reshape(n_r, out_dim, out_dim)
    pri = relation_pri.reshape(1, n_r)
    nt2 = node_type.reshape(n, 1)

    k_tab, qw_tab, vw_tab = _proj(nt2, h, kw, qw, vw, ratt, rmsg, pri)

    ecols = 128
    erows = e // ecols
    src2 = edge_index[0].reshape(erows, ecols)
    dst2 = edge_index[1].reshape(erows, ecols)
    et2 = edge_type.reshape(erows, ecols)
    info = plsc.get_sparse_core_info()
    nw = info.num_cores * info.num_subcores
    ept = e // nw
    epp = -(-ept // _C) * _C
    st = jnp.stack([a.reshape(e) for a in _edgeidx(src2, dst2, et2, n)])
    st = st.reshape(4, nw, ept)
    if epp > ept:
        dummy = jnp.concatenate(
            [jnp.zeros((3, nw, epp - ept), jnp.int32),
             jnp.full((1, nw, epp - ept), n, jnp.int32)], axis=0)
        st = jnp.concatenate([st, dummy], axis=2)
    eidx = st.reshape(4, nw, epp // _C, _C).transpose(1, 2, 0, 3).reshape(-1)

    agg2, denp = _edge_sc(qw_tab.reshape(n_r * n, out_dim), k_tab,
                          vw_tab.reshape(n_r * n, out_dim), eidx, n)
    den2 = denp.reshape(2, -1)[:, :n].reshape(2, n, 1)

    aw = a_linears.reshape(n_t, out_dim, out_dim)
    skip_row = skip.reshape(1, n_t)
    return _final(nt2, agg2, den2, aw, skip_row)


# speedup vs baseline: 1.0176x; 1.0176x over previous
import functools
import math

import jax
import jax.numpy as jnp
from jax import lax
from jax.experimental import pallas as pl
from jax.experimental.pallas import tpu as pltpu
from jax.experimental.pallas import tpu_sc as plsc

_BN = 1000
_C = 48


def _proj_body(nt_ref, h_ref, kw_ref, qw_ref, vw_ref, ratt_ref, rmsg_ref,
               pri_ref, k_out, qwt_out, vwt_out):
    nt = nt_ref[...]
    h = h_ref[...]
    n_t = kw_ref.shape[0]
    n_r = ratt_ref.shape[0]
    k = jnp.zeros((h.shape[0], kw_ref.shape[2]), jnp.float32)
    q = jnp.zeros_like(k)
    v = jnp.zeros_like(k)
    for t in range(n_t):
        m = nt == t
        k = jnp.where(m, jnp.dot(h, kw_ref[t], preferred_element_type=jnp.float32), k)
        q = jnp.where(m, jnp.dot(h, qw_ref[t], preferred_element_type=jnp.float32), q)
        v = jnp.where(m, jnp.dot(h, vw_ref[t], preferred_element_type=jnp.float32), v)
    k_out[...] = k
    inv_sqrt_dk = 1.0 / math.sqrt(float(kw_ref.shape[2]))
    for r in range(n_r):
        pri = pri_ref[:, r:r + 1] * inv_sqrt_dk
        qwt_out[r] = jnp.dot(q, ratt_ref[r], preferred_element_type=jnp.float32) * pri
        vwt_out[r] = jnp.dot(v, rmsg_ref[r], preferred_element_type=jnp.float32)


def _proj(nt2, h, kw, qw, vw, ratt, rmsg, pri):
    n, in_dim = h.shape
    out_dim = kw.shape[2]
    n_t, n_r = kw.shape[0], ratt.shape[0]
    grid = (n // _BN,)
    return pl.pallas_call(
        _proj_body,
        grid=grid,
        in_specs=[
            pl.BlockSpec((_BN, 1), lambda i: (i, 0)),
            pl.BlockSpec((_BN, in_dim), lambda i: (i, 0)),
            pl.BlockSpec((n_t, in_dim, out_dim), lambda i: (0, 0, 0)),
            pl.BlockSpec((n_t, in_dim, out_dim), lambda i: (0, 0, 0)),
            pl.BlockSpec((n_t, in_dim, out_dim), lambda i: (0, 0, 0)),
            pl.BlockSpec((n_r, out_dim, out_dim), lambda i: (0, 0, 0)),
            pl.BlockSpec((n_r, out_dim, out_dim), lambda i: (0, 0, 0)),
            pl.BlockSpec((1, n_r), lambda i: (0, 0)),
        ],
        out_specs=[
            pl.BlockSpec((_BN, out_dim), lambda i: (i, 0)),
            pl.BlockSpec((n_r, _BN, out_dim), lambda i: (0, i, 0)),
            pl.BlockSpec((n_r, _BN, out_dim), lambda i: (0, i, 0)),
        ],
        out_shape=[
            jax.ShapeDtypeStruct((n, out_dim), jnp.float32),
            jax.ShapeDtypeStruct((n_r, n, out_dim), jnp.float32),
            jax.ShapeDtypeStruct((n_r, n, out_dim), jnp.float32),
        ],
    )(nt2, h, kw, qw, vw, ratt, rmsg, pri)


def _edgeidx_body(n_nodes, src_ref, dst_ref, et_ref, qwi_ref, ki_ref,
                  vwi_ref, di_ref):
    src = src_ref[...]
    dst = dst_ref[...]
    et = et_ref[...]
    qwi_ref[...] = et * n_nodes + dst
    ki_ref[...] = src
    vwi_ref[...] = et * n_nodes + src
    di_ref[...] = dst


def _edgeidx(src2, dst2, et2, n_nodes):
    rows, cols = src2.shape
    shp = jax.ShapeDtypeStruct((rows, cols), jnp.int32)
    return pl.pallas_call(
        functools.partial(_edgeidx_body, n_nodes),
        out_shape=[shp, shp, shp, shp],
    )(src2, dst2, et2)


def _edge_sc(qw_tab, k_tab, vw_tab, eidx, n_nodes):
    out_dim = k_tab.shape[1]
    info = plsc.get_sparse_core_info()
    nc, ns = info.num_cores, info.num_subcores
    nw = nc * ns
    nch = eidx.shape[0] // (nw * 4 * _C)
    nrd = 10
    rs = n_nodes // nrd
    zr = 10
    nzero = rs // zr
    ndr = (-(-n_nodes // out_dim) + 15) // 16 * 16
    mesh = plsc.VectorSubcoreMesh(core_axis_name="c", subcore_axis_name="s")

    @functools.partial(
        pl.kernel,
        out_type=[
            jax.ShapeDtypeStruct((nc, n_nodes, out_dim), jnp.float32),
            jax.ShapeDtypeStruct((nc, ndr, out_dim), jnp.float32),
        ],
        mesh=mesh,
        compiler_params=pltpu.CompilerParams(needs_layout_passes=False),
        scratch_types=[
            pltpu.VMEM((4, 2 * _C), jnp.int32),
            pltpu.VMEM((_C,), jnp.int32),
            pltpu.VMEM((_C,), jnp.int32),
            pltpu.VMEM((ndr,), jnp.int32),
            pltpu.VMEM((2 * _C, out_dim), jnp.float32),
            pltpu.VMEM((2 * _C, out_dim), jnp.float32),
            pltpu.VMEM((2 * _C, out_dim), jnp.float32),
            pltpu.VMEM((ndr * out_dim,), jnp.float32),
            pltpu.VMEM((_C,), jnp.float32),
            pltpu.VMEM((zr, out_dim), jnp.float32),
            pltpu.VMEM_SHARED((n_nodes + 8, out_dim), jnp.float32),
            pltpu.VMEM_SHARED((ndr, out_dim), jnp.float32),
            pltpu.SemaphoreType.DMA,
            pltpu.SemaphoreType.DMA,
            pltpu.SemaphoreType.DMA,
            pltpu.SemaphoreType.DMA,
            pltpu.SemaphoreType.DMA,
            pltpu.SemaphoreType.DMA,
        ],
    )
    def _k(qw_hbm, k_hbm, vw_hbm, eidx_hbm,
           agg_out, den_out,
           ibuf, sbuf0, sbuf1, idb, qwr, kr, vwr, denp, exv, zbuf, agg_sh,
           den_sh,
           semi0, semi1, semg0, semg1, sems0, sems1):
        cid = lax.axis_index("c")
        sid = lax.axis_index("s")
        wid = sid * nc + cid
        semi = (semi0, semi1)
        semg = (semg0, semg1)
        sems = (sems0, sems1)
        sbuf = (sbuf0, sbuf1)
        zf16 = jnp.zeros((16,), jnp.float32)
        iota16 = lax.iota(jnp.int32, 16)
        bfly = [(iota16 ^ sh)[:, None] for sh in (8, 4, 2, 1)]
        dn = lax.GatherDimensionNumbers(
            offset_dims=(), collapsed_slice_dims=(0,), start_index_map=(0,))

        def _hsum(x):
            for idx in bfly:
                x = x + lax.gather(x, idx, dn, slice_sizes=(1,),
                                   mode=lax.GatherScatterMode.PROMISE_IN_BOUNDS)
            return x

        def _zrow(i, _):
            r = i // (out_dim // 16)
            c = (i % (out_dim // 16)) * 16
            zbuf[r, pl.ds(c, 16)] = zf16
            return _
        lax.fori_loop(0, zr * (out_dim // 16), _zrow, None)

        def _zden(i, _):
            denp[pl.ds(i * 16, 16)] = zf16
            return _
        lax.fori_loop(0, ndr * out_dim // 16, _zden, None)

        for g in range(ndr // 16):
            idb[pl.ds(g * 16, 16)] = iota16 + g * 16

        @pl.when(sid < nrd)
        def _zero_shared():
            cps = [pltpu.async_copy(zbuf,
                                    agg_sh.at[pl.ds(sid * rs + g * zr, zr)],
                                    semg0)
                   for g in range(nzero)]
            for cp in cps:
                cp.wait()

        @pl.when(sid == nrd)
        def _zero_den():
            cps = [pltpu.async_copy(zbuf, den_sh.at[pl.ds(g * zr, zr)], semg0)
                   for g in range(ndr // zr)]
            for cp in cps:
                cp.wait()
        plsc.subcore_barrier()

        cbase = wid * nch

        def _idx_cps(g, b):
            base = (cbase + g) * (4 * _C)
            return [
                (eidx_hbm.at[pl.ds(base, 2 * _C)], ibuf.at[2 * b]),
                (eidx_hbm.at[pl.ds(base + 2 * _C, 2 * _C)], ibuf.at[2 * b + 1]),
            ]

        def _gath(b):
            base = b * _C
            return [
                (qw_hbm.at[ibuf.at[2 * b, pl.ds(0, _C)]],
                 qwr.at[pl.ds(base, _C)]),
                (k_hbm.at[ibuf.at[2 * b, pl.ds(_C, _C)]],
                 kr.at[pl.ds(base, _C)]),
                (vw_hbm.at[ibuf.at[2 * b + 1, pl.ds(0, _C)]],
                 vwr.at[pl.ds(base, _C)]),
            ]

        def _scat(b):
            return (vwr.at[pl.ds(b * _C, _C)], agg_sh.at[sbuf[b]])

        def _phase(g, b):
            nb = 1 - b

            @pl.when(g + 1 < nch)
            def _prefetch():
                for s, d in _idx_cps(g + 1, nb):
                    pltpu.make_async_copy(s, d, semi[nb]).wait()

                @pl.when(g >= 1)
                def _wait_scat():
                    s, d = _scat(nb)
                    pltpu.make_async_copy(s, d, sems[nb]).wait()
                for s, d in _gath(nb):
                    pltpu.async_copy(s, d, semg[nb])

            for s, d in _gath(b):
                pltpu.make_async_copy(s, d, semg[b]).wait()

            for grp in range(_C // 16):
                sbuf[b][pl.ds(grp * 16, 16)] = ibuf[2 * b + 1, pl.ds(_C + grp * 16, 16)]

            @pl.when(g + 2 < nch)
            def _prefetch_idx():
                for s, d in _idx_cps(g + 2, b):
                    pltpu.async_copy(s, d, semi[b])

            base = b * _C

            @plsc.parallel_loop(0, _C, 1, unroll=4)
            def _edge(i):
                e = base + i
                acc = qwr[e, pl.ds(0, 16)] * kr[e, pl.ds(0, 16)]
                for c in range(1, out_dim // 16):
                    acc = acc + qwr[e, pl.ds(c * 16, 16)] * kr[e, pl.ds(c * 16, 16)]
                exb = jnp.exp(_hsum(acc))
                plsc.store_scatter(exv, [jnp.full((16,), i, jnp.int32)], exb,
                                   mask=iota16 == 0)
                for c in range(out_dim // 16):
                    vwr[e, pl.ds(c * 16, 16)] = vwr[e, pl.ds(c * 16, 16)] * exb

            def _grp(grp, _):
                dstv = sbuf[b][pl.ds(grp * 16, 16)]
                exg = exv[pl.ds(grp * 16, 16)]

                def _den(j, _):
                    plsc.addupdate_scatter(denp, [dstv], exg, mask=iota16 == j)
                    return _
                lax.fori_loop(0, 16, _den, None)
                return _
            lax.fori_loop(0, _C // 16, _grp, None)
            s, d = _scat(b)
            pltpu.async_copy(s, d, sems[b], add=True)

        for s, d in _idx_cps(0, 0):
            pltpu.async_copy(s, d, semi0)
        for s, d in _idx_cps(0, 0):
            pltpu.make_async_copy(s, d, semi0).wait()
        for s, d in _gath(0):
            pltpu.async_copy(s, d, semg0)
        for s, d in _idx_cps(1, 1):
            pltpu.async_copy(s, d, semi1)

        def _outer(o, _):
            _phase(o * 2, 0)
            _phase(o * 2 + 1, 1)
            return _
        lax.fori_loop(0, nch // 2, _outer, None)
        _phase(nch - 1, 0)
        for b in (0, 1):
            s, d = _scat(b)
            pltpu.make_async_copy(s, d, sems[b]).wait()

        def _pack(i, _):
            r = i // (out_dim // 16)
            c = (i % (out_dim // 16)) * 16
            qwr[r, pl.ds(c, 16)] = denp[pl.ds(r * out_dim + c, 16)]
            return _
        lax.fori_loop(0, ndr * out_dim // 16, _pack, None)
        pltpu.sync_copy(qwr.at[pl.ds(0, ndr)], den_sh.at[idb], add=True)
        plsc.subcore_barrier()

        @pl.when(sid < nrd)
        def _readout():
            pltpu.sync_copy(agg_sh.at[pl.ds(sid * rs, rs)],
                            agg_out.at[cid, pl.ds(sid * rs, rs)])

        @pl.when(sid == nrd)
        def _readout_den():
            pltpu.sync_copy(den_sh, den_out.at[cid])

    return _k(qw_tab, k_tab, vw_tab, eidx)


def _final_body(nt_ref, agg_ref, den_ref, aw_ref, skip_ref, out_ref):
    nt = nt_ref[...]
    agg = agg_ref[0] + agg_ref[1]
    den = den_ref[0] + den_ref[1]
    rows = agg / (den + 1e-16)
    n_t = aw_ref.shape[0]
    out = jnp.zeros_like(rows)
    for t in range(n_t):
        sig = 1.0 / (1.0 + jnp.exp(-skip_ref[:, t:t + 1]))
        y = jnp.dot(rows, aw_ref[t], preferred_element_type=jnp.float32) * sig
        out = jnp.where(nt == t, y, out)
    out_ref[...] = out


def _final(nt2, agg2, den2, aw, skip_row):
    n = nt2.shape[0]
    out_dim = aw.shape[2]
    n_t = aw.shape[0]
    grid = (n // _BN,)
    return pl.pallas_call(
        _final_body,
        grid=grid,
        in_specs=[
            pl.BlockSpec((_BN, 1), lambda i: (i, 0)),
            pl.BlockSpec((2, _BN, out_dim), lambda i: (0, i, 0)),
            pl.BlockSpec((2, _BN, 1), lambda i: (0, i, 0)),
            pl.BlockSpec((n_t, out_dim, out_dim), lambda i: (0, 0, 0)),
            pl.BlockSpec((1, n_t), lambda i: (0, 0)),
        ],
        out_specs=pl.BlockSpec((_BN, out_dim), lambda i: (i, 0)),
        out_shape=jax.ShapeDtypeStruct((n, out_dim), jnp.float32),
    )(nt2, agg2, den2, aw, skip_row)


def kernel(h, edge_index, edge_type, node_type, k_linears, q_linears,
           v_linears, a_linears, relation_att, relation_msg, relation_pri,
           skip):
    n, in_dim = h.shape
    e = edge_index.shape[1]
    n_t = k_linears.shape[0]
    n_r = relation_att.shape[0]
    out_dim = k_linears.shape[3]

    kw = k_linears.reshape(n_t, in_dim, out_dim)
    qw = q_linears.reshape(n_t, in_dim, out_dim)
    vw = v_linears.reshape(n_t, in_dim, out_dim)
    ratt = relation_att.reshape(n_r, out_dim, out_dim)
    rmsg = relation_msg.reshape(n_r, out_dim, out_dim)
    pri = relation_pri.reshape(1, n_r)
    nt2 = node_type.reshape(n, 1)

    k_tab, qw_tab, vw_tab = _proj(nt2, h, kw, qw, vw, ratt, rmsg, pri)

    ecols = 128
    erows = e // ecols
    src2 = edge_index[0].reshape(erows, ecols)
    dst2 = edge_index[1].reshape(erows, ecols)
    et2 = edge_type.reshape(erows, ecols)
    info = plsc.get_sparse_core_info()
    nw = info.num_cores * info.num_subcores
    ept = e // nw
    epp = -(-ept // _C) * _C
    st = jnp.stack([a.reshape(e) for a in _edgeidx(src2, dst2, et2, n)])
    st = st.reshape(4, nw, ept)
    if epp > ept:
        dummy = jnp.concatenate(
            [jnp.zeros((3, nw, epp - ept), jnp.int32),
             jnp.full((1, nw, epp - ept), n, jnp.int32)], axis=0)
        st = jnp.concatenate([st, dummy], axis=2)
    eidx = st.reshape(4, nw, epp // _C, _C).transpose(1, 2, 0, 3).reshape(-1)

    agg2, denp = _edge_sc(qw_tab.reshape(n_r * n, out_dim), k_tab,
                          vw_tab.reshape(n_r * n, out_dim), eidx, n)
    den2 = denp.reshape(2, -1)[:, :n].reshape(2, n, 1)

    aw = a_linears.reshape(n_t, out_dim, out_dim)
    skip_row = skip.reshape(1, n_t)
    return _final(nt2, agg2, den2, aw, skip_row)
